# baseline (speedup 1.0000x reference)
import functools

import jax
import jax.numpy as jnp
from jax import lax
from jax.experimental import pallas as pl
from jax.experimental.pallas import tpu as pltpu
from jax.experimental.pallas import tpu_sc as plsc

N_NODES = 10000
N_EDGES = 320000
DIM = 128
NUM_REL = 8

_NC = 2
_NS = 16
_NW = _NC * _NS
_EPW = N_EDGES // _NW
_CH = 128
_NCH = _EPW // _CH
_TAIL = _EPW - _NCH * _CH
_NPC = N_NODES // _NS
_CNT = N_NODES * NUM_REL
_CSL = _CNT // _NS
_K = 128
_NBLK = 2560
_CPW = _NBLK // _NW
_EPAD = _NBLK * _K - N_EDGES

_mesh = plsc.VectorSubcoreMesh(core_axis_name="c", subcore_axis_name="s")


@functools.partial(
    pl.kernel,
    out_type=jax.ShapeDtypeStruct((_NC * _CNT,), jnp.float32),
    mesh=_mesh,
    scratch_types=[
        pltpu.VMEM((_CPW, _K), jnp.int32),
        pltpu.VMEM((128,), jnp.float32),
        pltpu.VMEM((_CSL + 8,), jnp.float32),
        pltpu.VMEM_SHARED((_CNT + 8,), jnp.float32),
        pltpu.SemaphoreType.DMA,
    ],
)
def _sc_counts(icnt_hbm, out_hbm, ibuf, ones_v, vb, acc, sem):
    cid = lax.axis_index("c")
    sid = lax.axis_index("s")
    wid = sid * _NC + cid

    def zgrp(i, carry):
        vb[pl.ds(i * 16, 16)] = jnp.zeros((16,), jnp.float32)
        return carry

    lax.fori_loop(0, (_CSL + 8) // 16, zgrp, 0)
    pltpu.sync_copy(vb.at[pl.ds(0, _CSL)], acc.at[pl.ds(sid * _CSL, _CSL)])
    for i in range(128 // 16):
        ones_v[pl.ds(i * 16, 16)] = jnp.full((16,), 1.0, jnp.float32)
    base_e = wid * _CPW * _K

    def fill(c, carry):
        pltpu.async_copy(icnt_hbm.at[pl.ds(base_e + c * _K, _K)],
                         ibuf.at[c], sem)
        return carry

    lax.fori_loop(0, _CPW, fill, 0)

    def fdrain(c, carry):
        pltpu.make_async_copy(icnt_hbm.at[pl.ds(0, _K)], ibuf.at[0],
                              sem).wait()
        return carry

    lax.fori_loop(0, _CPW, fdrain, 0)
    plsc.subcore_barrier()

    def fire(c, carry):
        pltpu.async_copy(ones_v, acc.at[ibuf.at[c]], sem, add=True)
        return carry

    lax.fori_loop(0, _CPW, fire, 0)

    def drain(c, carry):
        pltpu.make_async_copy(ones_v, acc.at[ibuf.at[0]], sem).wait()
        return carry

    lax.fori_loop(0, _CPW, drain, 0)
    plsc.subcore_barrier()
    pltpu.sync_copy(acc.at[pl.ds(sid * _CSL, _CSL)], vb.at[pl.ds(0, _CSL)])
    pltpu.sync_copy(vb.at[pl.ds(0, _CSL)],
                    out_hbm.at[pl.ds(cid * _CNT + sid * _CSL, _CSL)])


def _tc_inv(cnt2):
    def body(p_ref, o_ref):
        o_ref[...] = 1.0 / jnp.maximum(p_ref[0] + p_ref[1], 1.0)

    return pl.pallas_call(
        body,
        out_shape=jax.ShapeDtypeStruct((_CNT // DIM, DIM), jnp.float32),
    )(cnt2)


@functools.partial(
    pl.kernel,
    out_type=jax.ShapeDtypeStruct((_NC, N_NODES, DIM), jnp.float32),
    mesh=_mesh,
    scratch_types=[
        pltpu.VMEM((3, _K), jnp.int32),
        pltpu.VMEM((3, _K), jnp.int32),
        pltpu.VMEM((_K, DIM), jnp.float32),
        pltpu.VMEM((_K, DIM), jnp.float32),
        pltpu.VMEM((128,), jnp.float32),
        pltpu.VMEM((128,), jnp.float32),
        pltpu.VMEM_SHARED((N_NODES, DIM), jnp.float32),
        pltpu.SemaphoreType.DMA,
        pltpu.SemaphoreType.DMA,
        pltpu.SemaphoreType.DMA,
        pltpu.SemaphoreType.DMA,
        pltpu.SemaphoreType.DMA,
        pltpu.SemaphoreType.DMA,
    ],
)
def _sc_agg(ytab_hbm, idx3_hbm, inv_hbm, out_hbm,
            cb0, cb1, rb0, rb1, sb0, sb1, acc,
            si0, si1, sg0, sg1, ss0, ss1):
    cid = lax.axis_index("c")
    sid = lax.axis_index("s")
    wid = sid * _NC + cid
    cbs, sis = (cb0, cb1), (si0, si1)
    rbs, sbs = (rb0, rb1), (sb0, sb1)
    sgs, sss = (sg0, sg1), (ss0, ss1)

    def zgrp(i, carry):
        for j in range(DIM // 16):
            rb0[i, pl.ds(j * 16, 16)] = jnp.zeros((16,), jnp.float32)
        return carry

    lax.fori_loop(0, _K, zgrp, 0)

    def spread(r0, sizes, to_acc):
        off = 0
        for s in sizes:
            if to_acc:
                pltpu.sync_copy(rb0.at[pl.ds(0, s)],
                                acc.at[pl.ds(r0 + off, s)])
            else:
                pltpu.sync_copy(acc.at[pl.ds(r0 + off, s)],
                                rb0.at[pl.ds(0, s)])
                pltpu.sync_copy(rb0.at[pl.ds(0, s)],
                                out_hbm.at[cid, pl.ds(r0 + off, s)])
            off += s

    @pl.when(sid < _NS - 1)
    def _():
        spread(sid * 632, [120, 120, 120, 120, 120, 32], True)

    @pl.when(sid == _NS - 1)
    def _():
        spread((_NS - 1) * 632, [120, 120, 120, 120, 40], True)

    plsc.subcore_barrier()
    base_c = wid * _CPW

    def i_start(ci, c):
        pltpu.async_copy(idx3_hbm.at[base_c + c], cbs[ci], sis[ci])

    def i_wait(ci):
        pltpu.make_async_copy(idx3_hbm.at[0], cbs[ci], sis[ci]).wait()

    def g_start(b, ci):
        pltpu.async_copy(ytab_hbm.at[cbs[ci].at[0]], rbs[b], sgs[b])
        pltpu.async_copy(inv_hbm.at[cbs[ci].at[2]], sbs[b], sss[b])

    def g_wait(b):
        pltpu.make_async_copy(ytab_hbm.at[cbs[b].at[0]], rbs[b],
                              sgs[b]).wait()
        pltpu.make_async_copy(inv_hbm.at[cbs[b].at[2]], sbs[b],
                              sss[b]).wait()

    def scale_scatter(b, ci):
        rb, sb = rbs[b], sbs[b]

        def one(e, s):
            for j in range(DIM // 16):
                rb[e, pl.ds(j * 16, 16)] = rb[e, pl.ds(j * 16, 16)] * s

        def grp(g, carry):
            sv = sb[pl.ds(g * 16, 16)]
            for l in range(16):
                one(g * 16 + l, sv[l])
            return carry

        lax.fori_loop(0, _K // 16, grp, 0)
        pltpu.sync_copy(rb, acc.at[cbs[ci].at[1]], add=True)

    i_start(0, 0)
    i_wait(0)
    g_start(0, 0)

    def pair(p, carry):
        a = 2 * p
        i_start(1, a + 1)
        i_wait(1)
        g_start(1, 1)
        g_wait(0)
        scale_scatter(0, 0)
        i_start(0, a + 2)
        i_wait(0)
        g_start(0, 0)
        g_wait(1)
        scale_scatter(1, 1)
        return carry

    lax.fori_loop(0, _CPW // 2 - 1, pair, 0)
    i_start(1, _CPW - 1)
    i_wait(1)
    g_start(1, 1)
    g_wait(0)
    scale_scatter(0, 0)
    g_wait(1)
    scale_scatter(1, 1)

    plsc.subcore_barrier()

    @pl.when(sid < _NS - 1)
    def _():
        spread(sid * 632, [120, 120, 120, 120, 120, 32], False)

    @pl.when(sid == _NS - 1)
    def _():
        spread((_NS - 1) * 632, [120, 120, 120, 120, 40], False)


_BN = 1000


def _tc_transform(x, wc, b):
    nb = N_NODES // _BN

    def body(x_ref, w_ref, b_ref, o_ref):
        r = pl.program_id(0)
        y = jnp.dot(x_ref[...], w_ref[0], preferred_element_type=jnp.float32)
        o_ref[0] = y + jnp.where(r == NUM_REL, 1.0, 0.0) * b_ref[...]

    return pl.pallas_call(
        body,
        grid=(NUM_REL + 1, nb),
        in_specs=[
            pl.BlockSpec((_BN, DIM), lambda r, i: (i, 0)),
            pl.BlockSpec((1, DIM, DIM), lambda r, i: (r, 0, 0)),
            pl.BlockSpec((1, DIM), lambda r, i: (0, 0)),
        ],
        out_specs=pl.BlockSpec((1, _BN, DIM), lambda r, i: (r, i, 0)),
        out_shape=jax.ShapeDtypeStruct((NUM_REL + 1, N_NODES, DIM),
                                       jnp.float32),
    )(x, wc, b)


def _tc_relu_transform(z, q, wc, b):
    nb = N_NODES // _BN

    def body(z_ref, q_ref, w_ref, b_ref, o_ref):
        r = pl.program_id(0)
        h = jnp.maximum(z_ref[...] + q_ref[0] + q_ref[1], 0.0)
        y = jnp.dot(h, w_ref[0], preferred_element_type=jnp.float32)
        o_ref[0] = y + jnp.where(r == NUM_REL, 1.0, 0.0) * b_ref[...]

    return pl.pallas_call(
        body,
        grid=(NUM_REL + 1, nb),
        in_specs=[
            pl.BlockSpec((_BN, DIM), lambda r, i: (i, 0)),
            pl.BlockSpec((_NC, _BN, DIM), lambda r, i: (0, i, 0)),
            pl.BlockSpec((1, DIM, DIM), lambda r, i: (r, 0, 0)),
            pl.BlockSpec((1, DIM), lambda r, i: (0, 0)),
        ],
        out_specs=pl.BlockSpec((1, _BN, DIM), lambda r, i: (r, i, 0)),
        out_shape=jax.ShapeDtypeStruct((NUM_REL + 1, N_NODES, DIM),
                                       jnp.float32),
    )(z, q, wc, b)


def _tc_final(z, q):
    nb = N_NODES // _BN

    def body(z_ref, q_ref, o_ref):
        o_ref[...] = z_ref[...] + q_ref[0] + q_ref[1]

    return pl.pallas_call(
        body,
        grid=(nb,),
        in_specs=[
            pl.BlockSpec((_BN, DIM), lambda i: (i, 0)),
            pl.BlockSpec((_NC, _BN, DIM), lambda i: (0, i, 0)),
        ],
        out_specs=pl.BlockSpec((_BN, DIM), lambda i: (i, 0)),
        out_shape=jax.ShapeDtypeStruct((N_NODES, DIM), jnp.float32),
    )(z, q)


def kernel(node_features, edge_index, edge_type, W1, root1, b1, W2, root2, b2):
    x = node_features
    src = edge_index[0].astype(jnp.int32)
    dst = edge_index[1].astype(jnp.int32)
    et = edge_type.astype(jnp.int32)
    isrc = et * N_NODES + src
    icnt = dst * NUM_REL + et

    zpad = jnp.zeros((_EPAD,), jnp.int32)
    isrc1 = jnp.concatenate([isrc, zpad])
    idst1 = jnp.concatenate([dst, zpad])
    icnt1 = jnp.concatenate([icnt, jnp.full((_EPAD,), _CNT, jnp.int32)])
    idx3 = jnp.stack([isrc1.reshape(-1, _K), idst1.reshape(-1, _K),
                      icnt1.reshape(-1, _K)], axis=1)

    cnt_part = _sc_counts(icnt1)
    inv = _tc_inv(cnt_part.reshape(_NC, _CNT // DIM, DIM)).reshape(_CNT)
    inv = jnp.concatenate([inv, jnp.zeros((8,), jnp.float32)])

    wc1 = jnp.concatenate([W1, root1[None]], axis=0)
    wc2 = jnp.concatenate([W2, root2[None]], axis=0)

    y1 = _tc_transform(x, wc1, b1.reshape(1, DIM))
    q1 = _sc_agg(y1.reshape(-1, DIM), idx3, inv)
    y2 = _tc_relu_transform(y1[NUM_REL], q1, wc2, b2.reshape(1, DIM))
    q2 = _sc_agg(y2.reshape(-1, DIM), idx3, inv)
    return _tc_final(y2[NUM_REL], q2)


# speedup vs baseline: 1.8335x; 1.8335x over previous
import functools

import jax
import jax.numpy as jnp
from jax import lax
from jax.experimental import pallas as pl
from jax.experimental.pallas import tpu as pltpu
from jax.experimental.pallas import tpu_sc as plsc

N_NODES = 10000
N_EDGES = 320000
DIM = 128
NUM_REL = 8

_NC = 2
_NS = 16
_NW = _NC * _NS
_EPW = N_EDGES // _NW
_CH = 128
_NCH = _EPW // _CH
_TAIL = _EPW - _NCH * _CH
_NPC = N_NODES // _NS
_CNT = N_NODES * NUM_REL
_CSL = _CNT // _NS
_K = 128
_NBLK = 2560
_CPW = _NBLK // _NW
_EPAD = _NBLK * _K - N_EDGES

_mesh = plsc.VectorSubcoreMesh(core_axis_name="c", subcore_axis_name="s")


@functools.partial(
    pl.kernel,
    out_type=jax.ShapeDtypeStruct((_NC * _CNT,), jnp.float32),
    mesh=_mesh,
    scratch_types=[
        pltpu.VMEM((_CPW, _K), jnp.int32),
        pltpu.VMEM((128,), jnp.float32),
        pltpu.VMEM((_CSL + 8,), jnp.float32),
        pltpu.VMEM_SHARED((_CNT + 8,), jnp.float32),
        pltpu.SemaphoreType.DMA,
    ],
)
def _sc_counts(icnt_hbm, out_hbm, ibuf, ones_v, vb, acc, sem):
    cid = lax.axis_index("c")
    sid = lax.axis_index("s")
    wid = sid * _NC + cid

    def zgrp(i, carry):
        vb[pl.ds(i * 16, 16)] = jnp.zeros((16,), jnp.float32)
        return carry

    lax.fori_loop(0, (_CSL + 8) // 16, zgrp, 0)
    pltpu.sync_copy(vb.at[pl.ds(0, _CSL)], acc.at[pl.ds(sid * _CSL, _CSL)])
    for i in range(128 // 16):
        ones_v[pl.ds(i * 16, 16)] = jnp.full((16,), 1.0, jnp.float32)
    base_e = wid * _CPW * _K

    def fill(c, carry):
        pltpu.async_copy(icnt_hbm.at[pl.ds(base_e + c * _K, _K)],
                         ibuf.at[c], sem)
        return carry

    lax.fori_loop(0, _CPW, fill, 0)

    def fdrain(c, carry):
        pltpu.make_async_copy(icnt_hbm.at[pl.ds(0, _K)], ibuf.at[0],
                              sem).wait()
        return carry

    lax.fori_loop(0, _CPW, fdrain, 0)
    plsc.subcore_barrier()

    def fire(c, carry):
        pltpu.async_copy(ones_v, acc.at[ibuf.at[c]], sem, add=True)
        return carry

    lax.fori_loop(0, _CPW, fire, 0)

    def drain(c, carry):
        pltpu.make_async_copy(ones_v, acc.at[ibuf.at[0]], sem).wait()
        return carry

    lax.fori_loop(0, _CPW, drain, 0)
    plsc.subcore_barrier()
    pltpu.sync_copy(acc.at[pl.ds(sid * _CSL, _CSL)], vb.at[pl.ds(0, _CSL)])
    pltpu.sync_copy(vb.at[pl.ds(0, _CSL)],
                    out_hbm.at[pl.ds(cid * _CNT + sid * _CSL, _CSL)])


def _tc_inv(cnt2):
    def body(p_ref, o_ref):
        o_ref[...] = 1.0 / jnp.maximum(p_ref[0] + p_ref[1], 1.0)

    return pl.pallas_call(
        body,
        out_shape=jax.ShapeDtypeStruct((_CNT // DIM, DIM), jnp.float32),
    )(cnt2)


@functools.partial(
    pl.kernel,
    out_type=jax.ShapeDtypeStruct((_NC, N_NODES, DIM), jnp.float32),
    mesh=_mesh,
    scratch_types=[
        pltpu.VMEM((3, _K), jnp.int32),
        pltpu.VMEM((3, _K), jnp.int32),
        pltpu.VMEM((_K, DIM), jnp.float32),
        pltpu.VMEM((_K, DIM), jnp.float32),
        pltpu.VMEM((128,), jnp.float32),
        pltpu.VMEM((128,), jnp.float32),
        pltpu.VMEM_SHARED((N_NODES, DIM), jnp.float32),
        pltpu.SemaphoreType.DMA,
        pltpu.SemaphoreType.DMA,
        pltpu.SemaphoreType.DMA,
        pltpu.SemaphoreType.DMA,
        pltpu.SemaphoreType.DMA,
        pltpu.SemaphoreType.DMA,
    ],
)
def _sc_agg(ytab_hbm, idx3_hbm, inv_hbm, out_hbm,
            cb0, cb1, rb0, rb1, sb0, sb1, acc,
            si0, si1, sg0, sg1, ss0, ss1):
    cid = lax.axis_index("c")
    sid = lax.axis_index("s")
    wid = sid * _NC + cid
    cbs, sis = (cb0, cb1), (si0, si1)
    rbs, sbs = (rb0, rb1), (sb0, sb1)
    sgs, sss = (sg0, sg1), (ss0, ss1)

    def zgrp(i, carry):
        for j in range(DIM // 16):
            rb0[i, pl.ds(j * 16, 16)] = jnp.zeros((16,), jnp.float32)
        return carry

    lax.fori_loop(0, _K, zgrp, 0)

    def spread(r0, sizes, to_acc):
        off = 0
        for s in sizes:
            if to_acc:
                pltpu.sync_copy(rb0.at[pl.ds(0, s)],
                                acc.at[pl.ds(r0 + off, s)])
            else:
                pltpu.sync_copy(acc.at[pl.ds(r0 + off, s)],
                                rb0.at[pl.ds(0, s)])
                pltpu.sync_copy(rb0.at[pl.ds(0, s)],
                                out_hbm.at[cid, pl.ds(r0 + off, s)])
            off += s

    @pl.when(sid < _NS - 1)
    def _():
        spread(sid * 632, [120, 120, 120, 120, 120, 32], True)

    @pl.when(sid == _NS - 1)
    def _():
        spread((_NS - 1) * 632, [120, 120, 120, 120, 40], True)

    plsc.subcore_barrier()
    base_c = wid * _CPW

    def i_start(ci, c):
        pltpu.async_copy(idx3_hbm.at[base_c + c], cbs[ci], sis[ci])

    def i_wait(ci):
        pltpu.make_async_copy(idx3_hbm.at[0], cbs[ci], sis[ci]).wait()

    def g_start(b, ci):
        pltpu.async_copy(ytab_hbm.at[cbs[ci].at[0]], rbs[b], sgs[b])
        pltpu.async_copy(inv_hbm.at[cbs[ci].at[2]], sbs[b], sss[b])

    def g_wait(b):
        pltpu.make_async_copy(ytab_hbm.at[cbs[b].at[0]], rbs[b],
                              sgs[b]).wait()
        pltpu.make_async_copy(inv_hbm.at[cbs[b].at[2]], sbs[b],
                              sss[b]).wait()

    def scale_scatter(b, ci):
        rb, sb = rbs[b], sbs[b]

        def one(e, s):
            for j in range(DIM // 16):
                rb[e, pl.ds(j * 16, 16)] = rb[e, pl.ds(j * 16, 16)] * s

        def grp(g, carry):
            sv = sb[pl.ds(g * 16, 16)]
            for l in range(16):
                one(g * 16 + l, sv[l])
            return carry

        lax.fori_loop(0, _K // 16, grp, 0)
        pltpu.sync_copy(rb, acc.at[cbs[ci].at[1]], add=True)

    i_start(0, 0)
    i_wait(0)
    g_start(0, 0)

    def pair(p, carry):
        a = 2 * p
        i_start(1, a + 1)
        i_wait(1)
        g_start(1, 1)
        g_wait(0)
        scale_scatter(0, 0)
        i_start(0, a + 2)
        i_wait(0)
        g_start(0, 0)
        g_wait(1)
        scale_scatter(1, 1)
        return carry

    lax.fori_loop(0, _CPW // 2 - 1, pair, 0)
    i_start(1, _CPW - 1)
    i_wait(1)
    g_start(1, 1)
    g_wait(0)
    scale_scatter(0, 0)
    g_wait(1)
    scale_scatter(1, 1)

    plsc.subcore_barrier()

    @pl.when(sid < _NS - 1)
    def _():
        spread(sid * 632, [120, 120, 120, 120, 120, 32], False)

    @pl.when(sid == _NS - 1)
    def _():
        spread((_NS - 1) * 632, [120, 120, 120, 120, 40], False)


_BN = 1000


def _tc_transform(x, wc, b):
    nb = N_NODES // _BN

    def body(x_ref, w_ref, b_ref, o_ref):
        r = pl.program_id(0)
        y = jnp.dot(x_ref[...], w_ref[0], preferred_element_type=jnp.float32)
        o_ref[0] = y + jnp.where(r == NUM_REL, 1.0, 0.0) * b_ref[...]

    return pl.pallas_call(
        body,
        grid=(NUM_REL + 1, nb),
        in_specs=[
            pl.BlockSpec((_BN, DIM), lambda r, i: (i, 0)),
            pl.BlockSpec((1, DIM, DIM), lambda r, i: (r, 0, 0)),
            pl.BlockSpec((1, DIM), lambda r, i: (0, 0)),
        ],
        out_specs=pl.BlockSpec((1, _BN, DIM), lambda r, i: (r, i, 0)),
        out_shape=jax.ShapeDtypeStruct((NUM_REL + 1, N_NODES, DIM),
                                       jnp.float32),
    )(x, wc, b)


def _tc_relu_transform(z, q, wc, b):
    nb = N_NODES // _BN

    def body(z_ref, q_ref, w_ref, b_ref, o_ref):
        r = pl.program_id(0)
        h = jnp.maximum(z_ref[...] + q_ref[0] + q_ref[1], 0.0)
        y = jnp.dot(h, w_ref[0], preferred_element_type=jnp.float32)
        o_ref[0] = y + jnp.where(r == NUM_REL, 1.0, 0.0) * b_ref[...]

    return pl.pallas_call(
        body,
        grid=(NUM_REL + 1, nb),
        in_specs=[
            pl.BlockSpec((_BN, DIM), lambda r, i: (i, 0)),
            pl.BlockSpec((_NC, _BN, DIM), lambda r, i: (0, i, 0)),
            pl.BlockSpec((1, DIM, DIM), lambda r, i: (r, 0, 0)),
            pl.BlockSpec((1, DIM), lambda r, i: (0, 0)),
        ],
        out_specs=pl.BlockSpec((1, _BN, DIM), lambda r, i: (r, i, 0)),
        out_shape=jax.ShapeDtypeStruct((NUM_REL + 1, N_NODES, DIM),
                                       jnp.float32),
    )(z, q, wc, b)


def _tc_final(z, q):
    nb = N_NODES // _BN

    def body(z_ref, q_ref, o_ref):
        o_ref[...] = z_ref[...] + q_ref[0] + q_ref[1]

    return pl.pallas_call(
        body,
        grid=(nb,),
        in_specs=[
            pl.BlockSpec((_BN, DIM), lambda i: (i, 0)),
            pl.BlockSpec((_NC, _BN, DIM), lambda i: (0, i, 0)),
        ],
        out_specs=pl.BlockSpec((_BN, DIM), lambda i: (i, 0)),
        out_shape=jax.ShapeDtypeStruct((N_NODES, DIM), jnp.float32),
    )(z, q)


def kernel(node_features, edge_index, edge_type, W1, root1, b1, W2, root2, b2):
    x = node_features
    src = edge_index[0].astype(jnp.int32)
    dst = edge_index[1].astype(jnp.int32)
    et = edge_type.astype(jnp.int32)
    isrc = et * N_NODES + src
    icnt = dst * NUM_REL + et

    spad = jnp.arange(_EPAD, dtype=jnp.int32)
    isrc1 = jnp.concatenate([isrc, spad % (NUM_REL * N_NODES)])
    idst1 = jnp.concatenate([dst, spad % N_NODES])
    icnt1 = jnp.concatenate([icnt, _CNT + (spad % 8)])
    idx3 = jnp.stack([isrc1.reshape(-1, _K), idst1.reshape(-1, _K),
                      icnt1.reshape(-1, _K)], axis=1)

    cnt_part = _sc_counts(icnt1)
    inv = _tc_inv(cnt_part.reshape(_NC, _CNT // DIM, DIM)).reshape(_CNT)
    inv = jnp.concatenate([inv, jnp.zeros((8,), jnp.float32)])

    wc1 = jnp.concatenate([W1, root1[None]], axis=0)
    wc2 = jnp.concatenate([W2, root2[None]], axis=0)

    y1 = _tc_transform(x, wc1, b1.reshape(1, DIM))
    q1 = _sc_agg(y1.reshape(-1, DIM), idx3, inv)
    y2 = _tc_relu_transform(y1[NUM_REL], q1, wc2, b2.reshape(1, DIM))
    q2 = _sc_agg(y2.reshape(-1, DIM), idx3, inv)
    return _tc_final(y2[NUM_REL], q2)


# speedup vs baseline: 2.0295x; 1.1069x over previous
import functools

import jax
import jax.numpy as jnp
from jax import lax
from jax.experimental import pallas as pl
from jax.experimental.pallas import tpu as pltpu
from jax.experimental.pallas import tpu_sc as plsc

N_NODES = 10000
N_EDGES = 320000
DIM = 128
NUM_REL = 8

_NC = 2
_NS = 16
_NW = _NC * _NS
_EPW = N_EDGES // _NW
_CH = 128
_NCH = _EPW // _CH
_TAIL = _EPW - _NCH * _CH
_NPC = N_NODES // _NS
_CNT = N_NODES * NUM_REL
_CSL = _CNT // _NS
_K = 128
_NBLK = 2560
_CPW = _NBLK // _NW
_EPAD = _NBLK * _K - N_EDGES

_mesh = plsc.VectorSubcoreMesh(core_axis_name="c", subcore_axis_name="s")


@functools.partial(
    pl.kernel,
    out_type=jax.ShapeDtypeStruct((_NC * _CNT,), jnp.float32),
    mesh=_mesh,
    scratch_types=[
        pltpu.VMEM((_CPW, _K), jnp.int32),
        pltpu.VMEM((128,), jnp.float32),
        pltpu.VMEM((_CSL + 8,), jnp.float32),
        pltpu.VMEM_SHARED((_CNT + 8,), jnp.float32),
        pltpu.SemaphoreType.DMA,
    ],
)
def _sc_counts(icnt_hbm, out_hbm, ibuf, ones_v, vb, acc, sem):
    cid = lax.axis_index("c")
    sid = lax.axis_index("s")
    wid = sid * _NC + cid

    def zgrp(i, carry):
        vb[pl.ds(i * 16, 16)] = jnp.zeros((16,), jnp.float32)
        return carry

    lax.fori_loop(0, (_CSL + 8) // 16, zgrp, 0)
    pltpu.sync_copy(vb.at[pl.ds(0, _CSL)], acc.at[pl.ds(sid * _CSL, _CSL)])
    for i in range(128 // 16):
        ones_v[pl.ds(i * 16, 16)] = jnp.full((16,), 1.0, jnp.float32)
    base_e = wid * _CPW * _K

    def fill(c, carry):
        pltpu.async_copy(icnt_hbm.at[pl.ds(base_e + c * _K, _K)],
                         ibuf.at[c], sem)
        return carry

    lax.fori_loop(0, _CPW, fill, 0)

    def fdrain(c, carry):
        pltpu.make_async_copy(icnt_hbm.at[pl.ds(0, _K)], ibuf.at[0],
                              sem).wait()
        return carry

    lax.fori_loop(0, _CPW, fdrain, 0)
    plsc.subcore_barrier()

    def fire(c, carry):
        pltpu.async_copy(ones_v, acc.at[ibuf.at[c]], sem, add=True)
        return carry

    lax.fori_loop(0, _CPW, fire, 0)

    def drain(c, carry):
        pltpu.make_async_copy(ones_v, acc.at[ibuf.at[0]], sem).wait()
        return carry

    lax.fori_loop(0, _CPW, drain, 0)
    plsc.subcore_barrier()
    pltpu.sync_copy(acc.at[pl.ds(sid * _CSL, _CSL)], vb.at[pl.ds(0, _CSL)])
    pltpu.sync_copy(vb.at[pl.ds(0, _CSL)],
                    out_hbm.at[pl.ds(cid * _CNT + sid * _CSL, _CSL)])


def _tc_inv(cnt2):
    def body(p_ref, o_ref):
        o_ref[...] = 1.0 / jnp.maximum(p_ref[0] + p_ref[1], 1.0)

    return pl.pallas_call(
        body,
        out_shape=jax.ShapeDtypeStruct((_CNT // DIM, DIM), jnp.float32),
    )(cnt2)


@functools.partial(
    pl.kernel,
    out_type=jax.ShapeDtypeStruct((_NC, N_NODES, DIM), jnp.float32),
    mesh=_mesh,
    scratch_types=[
        pltpu.VMEM((3, _K), jnp.int32),
        pltpu.VMEM((3, _K), jnp.int32),
        pltpu.VMEM((3, _K), jnp.int32),
        pltpu.VMEM((3, _K), jnp.int32),
        pltpu.VMEM((_K, DIM), jnp.float32),
        pltpu.VMEM((_K, DIM), jnp.float32),
        pltpu.VMEM((128,), jnp.float32),
        pltpu.VMEM((128,), jnp.float32),
        pltpu.VMEM_SHARED((N_NODES, DIM), jnp.float32),
        pltpu.SemaphoreType.DMA,
        pltpu.SemaphoreType.DMA,
        pltpu.SemaphoreType.DMA,
        pltpu.SemaphoreType.DMA,
        pltpu.SemaphoreType.DMA,
        pltpu.SemaphoreType.DMA,
        pltpu.SemaphoreType.DMA,
        pltpu.SemaphoreType.DMA,
    ],
)
def _sc_agg(ytab_hbm, idx3_hbm, inv_hbm, out_hbm,
            cb0, cb1, cb2, cb3, rb0, rb1, sb0, sb1, acc,
            si0, si1, si2, si3, sg0, sg1, ss0, ss1):
    cid = lax.axis_index("c")
    sid = lax.axis_index("s")
    wid = sid * _NC + cid
    cbs, sis = (cb0, cb1, cb2, cb3), (si0, si1, si2, si3)
    rbs, sbs = (rb0, rb1), (sb0, sb1)
    sgs, sss = (sg0, sg1), (ss0, ss1)

    def zgrp(i, carry):
        for j in range(DIM // 16):
            rb0[i, pl.ds(j * 16, 16)] = jnp.zeros((16,), jnp.float32)
        return carry

    lax.fori_loop(0, _K, zgrp, 0)

    def spread(r0, sizes, to_acc):
        off = 0
        for s in sizes:
            if to_acc:
                pltpu.sync_copy(rb0.at[pl.ds(0, s)],
                                acc.at[pl.ds(r0 + off, s)])
            else:
                pltpu.sync_copy(acc.at[pl.ds(r0 + off, s)],
                                rb0.at[pl.ds(0, s)])
                pltpu.sync_copy(rb0.at[pl.ds(0, s)],
                                out_hbm.at[cid, pl.ds(r0 + off, s)])
            off += s

    @pl.when(sid < _NS - 1)
    def _():
        spread(sid * 632, [120, 120, 120, 120, 120, 32], True)

    @pl.when(sid == _NS - 1)
    def _():
        spread((_NS - 1) * 632, [120, 120, 120, 120, 40], True)

    plsc.subcore_barrier()
    base_c = wid * _CPW

    def i_start(ci, c):
        pltpu.async_copy(idx3_hbm.at[base_c + c], cbs[ci], sis[ci])

    def i_wait(ci):
        pltpu.make_async_copy(idx3_hbm.at[0], cbs[ci], sis[ci]).wait()

    def g_start(b, ci):
        pltpu.async_copy(ytab_hbm.at[cbs[ci].at[0]], rbs[b], sgs[b])
        pltpu.async_copy(inv_hbm.at[cbs[ci].at[2]], sbs[b], sss[b])

    def g_wait(b):
        pltpu.make_async_copy(ytab_hbm.at[cbs[b].at[0]], rbs[b],
                              sgs[b]).wait()
        pltpu.make_async_copy(inv_hbm.at[cbs[b].at[2]], sbs[b],
                              sss[b]).wait()

    def scale_scatter(b, ci):
        rb, sb = rbs[b], sbs[b]

        def one(e, s):
            for j in range(DIM // 16):
                rb[e, pl.ds(j * 16, 16)] = rb[e, pl.ds(j * 16, 16)] * s

        def grp(g, carry):
            sv = sb[pl.ds(g * 16, 16)]
            for l in range(16):
                one(g * 16 + l, sv[l])
            return carry

        lax.fori_loop(0, _K // 16, grp, 0)
        pltpu.sync_copy(rb, acc.at[cbs[ci].at[1]], add=True)

    def step(k, kk, has_i, has_g):
        if has_i:
            i_start((kk + 3) % 4, k + 3)
        g_wait(kk % 2)
        scale_scatter(kk % 2, kk % 4)
        if has_g:
            i_wait((kk + 2) % 4)
            g_start(kk % 2, (kk + 2) % 4)

    i_start(0, 0)
    i_start(1, 1)
    i_start(2, 2)
    i_wait(0)
    g_start(0, 0)
    i_wait(1)
    g_start(1, 1)

    def quad(p, carry):
        a = 4 * p
        for j in range(4):
            step(a + j, j, True, True)
        return carry

    lax.fori_loop(0, _CPW // 4 - 1, quad, 0)
    step(_CPW - 4, 0, True, True)
    step(_CPW - 3, 1, False, True)
    step(_CPW - 2, 2, False, False)
    step(_CPW - 1, 3, False, False)

    plsc.subcore_barrier()

    @pl.when(sid < _NS - 1)
    def _():
        spread(sid * 632, [120, 120, 120, 120, 120, 32], False)

    @pl.when(sid == _NS - 1)
    def _():
        spread((_NS - 1) * 632, [120, 120, 120, 120, 40], False)


_BN = 1000


def _tc_transform(x, wc, b):
    nb = N_NODES // _BN

    def body(x_ref, w_ref, b_ref, o_ref):
        r = pl.program_id(0)
        y = jnp.dot(x_ref[...], w_ref[0], preferred_element_type=jnp.float32)
        o_ref[0] = y + jnp.where(r == NUM_REL, 1.0, 0.0) * b_ref[...]

    return pl.pallas_call(
        body,
        grid=(NUM_REL + 1, nb),
        in_specs=[
            pl.BlockSpec((_BN, DIM), lambda r, i: (i, 0)),
            pl.BlockSpec((1, DIM, DIM), lambda r, i: (r, 0, 0)),
            pl.BlockSpec((1, DIM), lambda r, i: (0, 0)),
        ],
        out_specs=pl.BlockSpec((1, _BN, DIM), lambda r, i: (r, i, 0)),
        out_shape=jax.ShapeDtypeStruct((NUM_REL + 1, N_NODES, DIM),
                                       jnp.float32),
    )(x, wc, b)


def _tc_relu_transform(z, q, wc, b):
    nb = N_NODES // _BN

    def body(z_ref, q_ref, w_ref, b_ref, o_ref):
        r = pl.program_id(0)
        h = jnp.maximum(z_ref[...] + q_ref[0] + q_ref[1], 0.0)
        y = jnp.dot(h, w_ref[0], preferred_element_type=jnp.float32)
        o_ref[0] = y + jnp.where(r == NUM_REL, 1.0, 0.0) * b_ref[...]

    return pl.pallas_call(
        body,
        grid=(NUM_REL + 1, nb),
        in_specs=[
            pl.BlockSpec((_BN, DIM), lambda r, i: (i, 0)),
            pl.BlockSpec((_NC, _BN, DIM), lambda r, i: (0, i, 0)),
            pl.BlockSpec((1, DIM, DIM), lambda r, i: (r, 0, 0)),
            pl.BlockSpec((1, DIM), lambda r, i: (0, 0)),
        ],
        out_specs=pl.BlockSpec((1, _BN, DIM), lambda r, i: (r, i, 0)),
        out_shape=jax.ShapeDtypeStruct((NUM_REL + 1, N_NODES, DIM),
                                       jnp.float32),
    )(z, q, wc, b)


def _tc_final(z, q):
    nb = N_NODES // _BN

    def body(z_ref, q_ref, o_ref):
        o_ref[...] = z_ref[...] + q_ref[0] + q_ref[1]

    return pl.pallas_call(
        body,
        grid=(nb,),
        in_specs=[
            pl.BlockSpec((_BN, DIM), lambda i: (i, 0)),
            pl.BlockSpec((_NC, _BN, DIM), lambda i: (0, i, 0)),
        ],
        out_specs=pl.BlockSpec((_BN, DIM), lambda i: (i, 0)),
        out_shape=jax.ShapeDtypeStruct((N_NODES, DIM), jnp.float32),
    )(z, q)


def kernel(node_features, edge_index, edge_type, W1, root1, b1, W2, root2, b2):
    x = node_features
    src = edge_index[0].astype(jnp.int32)
    dst = edge_index[1].astype(jnp.int32)
    et = edge_type.astype(jnp.int32)
    isrc = et * N_NODES + src
    icnt = dst * NUM_REL + et

    spad = jnp.arange(_EPAD, dtype=jnp.int32)
    isrc1 = jnp.concatenate([isrc, spad % (NUM_REL * N_NODES)])
    idst1 = jnp.concatenate([dst, spad % N_NODES])
    icnt1 = jnp.concatenate([icnt, _CNT + (spad % 8)])
    idx3 = jnp.stack([isrc1.reshape(-1, _K), idst1.reshape(-1, _K),
                      icnt1.reshape(-1, _K)], axis=1)

    cnt_part = _sc_counts(icnt1)
    inv = _tc_inv(cnt_part.reshape(_NC, _CNT // DIM, DIM)).reshape(_CNT)
    inv = jnp.concatenate([inv, jnp.zeros((8,), jnp.float32)])

    wc1 = jnp.concatenate([W1, root1[None]], axis=0)
    wc2 = jnp.concatenate([W2, root2[None]], axis=0)

    y1 = _tc_transform(x, wc1, b1.reshape(1, DIM))
    q1 = _sc_agg(y1.reshape(-1, DIM), idx3, inv)
    y2 = _tc_relu_transform(y1[NUM_REL], q1, wc2, b2.reshape(1, DIM))
    q2 = _sc_agg(y2.reshape(-1, DIM), idx3, inv)
    return _tc_final(y2[NUM_REL], q2)
